# baseline (speedup 1.0000x reference)
import functools

import jax
import jax.numpy as jnp
from jax import lax
from jax.experimental import pallas as pl
from jax.experimental.pallas import tpu as pltpu
from jax.experimental.pallas import tpu_sc as plsc

D = 1024
L = 16
NSLICE = D // L
NC, NS = 2, 16
NW = NC * NS
R = 8192
ROWS_PER_W = R // NW
CR = 16
NCH = ROWS_PER_W // CR


def _sc_kernel_body(lang_hbm, frames_hbm, actions_hbm, emb_hbm,
                    out_l, out_f, out_a,
                    emb_v, in0, in1, ou0, ou1,
                    si0, si1, so0, so1):
    wid = lax.axis_index("s") * NC + lax.axis_index("c")
    base = wid * ROWS_PER_W

    pltpu.sync_copy(emb_hbm, emb_v)

    in_bufs = (in0, in1)
    out_bufs = (ou0, ou1)
    in_sems = (si0, si1)
    out_sems = (so0, so1)

    def start_in(in_hbm, c, b):
        pltpu.make_async_copy(
            in_hbm.at[pl.ds(base + c * CR, CR)], in_bufs[b], in_sems[b]
        ).start()

    def wait_in(in_hbm, b):
        pltpu.make_async_copy(
            in_hbm.at[pl.ds(base, CR)], in_bufs[b], in_sems[b]
        ).wait()

    def start_out(out_hbm, c, b):
        pltpu.make_async_copy(
            out_bufs[b], out_hbm.at[pl.ds(base + c * CR, CR)], out_sems[b]
        ).start()

    def wait_out(out_hbm, b):
        pltpu.make_async_copy(
            out_bufs[b], out_hbm.at[pl.ds(base, CR)], out_sems[b]
        ).wait()

    def compute(b, t):
        ib, ob = in_bufs[b], out_bufs[b]

        def row_body(r, carry):
            for j in range(NSLICE):
                sl = pl.ds(j * L, L)
                ob[r, sl] = ib[r, sl] + emb_v[t, sl]
            return carry

        lax.fori_loop(0, CR, row_body, 0)

    def phase(in_hbm, out_hbm, t):
        start_in(in_hbm, 0, 0)
        start_in(in_hbm, 1, 1)

        def pair_body(p, carry):
            for b in range(2):
                c = 2 * p + b
                wait_in(in_hbm, b)

                @pl.when(p > 0)
                def _():
                    wait_out(out_hbm, b)

                compute(b, t)
                start_out(out_hbm, c, b)
                cn = jnp.minimum(c + 2, NCH - 1)
                start_in(in_hbm, cn, b)
            return carry

        lax.fori_loop(0, NCH // 2, pair_body, 0)
        wait_in(in_hbm, 0)
        wait_in(in_hbm, 1)
        wait_out(out_hbm, 0)
        wait_out(out_hbm, 1)

    phase(lang_hbm, out_l, 0)
    phase(frames_hbm, out_f, 1)
    phase(actions_hbm, out_a, 2)


def kernel(lang, frames, actions, emb_weight):
    B, S, Dm = lang.shape
    lf = lang.reshape(R, Dm)
    ff = frames.reshape(R, Dm)
    af = actions.reshape(R, Dm)

    mesh = plsc.VectorSubcoreMesh(core_axis_name="c", subcore_axis_name="s")
    f32 = jnp.float32
    sc_call = functools.partial(
        pl.kernel,
        mesh=mesh,
        out_type=[jax.ShapeDtypeStruct((R, Dm), f32)] * 3,
        scratch_types=[
            pltpu.VMEM((3, Dm), f32),
            pltpu.VMEM((CR, Dm), f32),
            pltpu.VMEM((CR, Dm), f32),
            pltpu.VMEM((CR, Dm), f32),
            pltpu.VMEM((CR, Dm), f32),
            pltpu.SemaphoreType.DMA,
            pltpu.SemaphoreType.DMA,
            pltpu.SemaphoreType.DMA,
            pltpu.SemaphoreType.DMA,
        ],
    )(_sc_kernel_body)
    out = sc_call(lf, ff, af, emb_weight)
    return tuple(o.reshape(B, S, Dm) for o in out)


# speedup vs baseline: 1.9950x; 1.9950x over previous
import functools

import jax
import jax.numpy as jnp
from jax import lax
from jax.experimental import pallas as pl
from jax.experimental.pallas import tpu as pltpu
from jax.experimental.pallas import tpu_sc as plsc

D = 1024
L = 16
NSLICE = D // L
NC, NS = 2, 16
NW = NC * NS
R = 8192
ROWS_PER_W = R // NW
CR = 16
NCH = ROWS_PER_W // CR


def _sc_kernel_body(lang_hbm, frames_hbm, actions_hbm, emb_hbm,
                    out_l, out_f, out_a,
                    emb_v, in0, in1, ou0, ou1,
                    si0, si1, so0, so1):
    wid = lax.axis_index("s") * NC + lax.axis_index("c")
    base = wid * ROWS_PER_W

    pltpu.sync_copy(emb_hbm, emb_v)

    in_bufs = (in0, in1)
    out_bufs = (ou0, ou1)
    in_sems = (si0, si1)
    out_sems = (so0, so1)

    def start_in(in_hbm, c, b):
        pltpu.make_async_copy(
            in_hbm.at[pl.ds(base + c * CR, CR)], in_bufs[b], in_sems[b]
        ).start()

    def wait_in(in_hbm, b):
        pltpu.make_async_copy(
            in_hbm.at[pl.ds(base, CR)], in_bufs[b], in_sems[b]
        ).wait()

    def start_out(out_hbm, c, b):
        pltpu.make_async_copy(
            out_bufs[b], out_hbm.at[pl.ds(base + c * CR, CR)], out_sems[b]
        ).start()

    def wait_out(out_hbm, b):
        pltpu.make_async_copy(
            out_bufs[b], out_hbm.at[pl.ds(base, CR)], out_sems[b]
        ).wait()

    def compute(b, t):
        ib, ob = in_bufs[b], out_bufs[b]
        GJ = 16
        for g in range(NSLICE // GJ):
            embs = [emb_v[t, pl.ds((g * GJ + k) * L, L)] for k in range(GJ)]

            @plsc.parallel_loop(0, CR, unroll=4)
            def _row(r):
                for k in range(GJ):
                    sl = pl.ds((g * GJ + k) * L, L)
                    ob[r, sl] = ib[r, sl] + embs[k]

    def phase(in_hbm, out_hbm, t):
        start_in(in_hbm, 0, 0)
        start_in(in_hbm, 1, 1)

        def pair_body(p, carry):
            for b in range(2):
                c = 2 * p + b
                wait_in(in_hbm, b)

                @pl.when(p > 0)
                def _():
                    wait_out(out_hbm, b)

                compute(b, t)
                start_out(out_hbm, c, b)
                cn = jnp.minimum(c + 2, NCH - 1)
                start_in(in_hbm, cn, b)
            return carry

        lax.fori_loop(0, NCH // 2, pair_body, 0)
        wait_in(in_hbm, 0)
        wait_in(in_hbm, 1)
        wait_out(out_hbm, 0)
        wait_out(out_hbm, 1)

    phase(lang_hbm, out_l, 0)
    phase(frames_hbm, out_f, 1)
    phase(actions_hbm, out_a, 2)


def kernel(lang, frames, actions, emb_weight):
    B, S, Dm = lang.shape
    lf = lang.reshape(R, Dm)
    ff = frames.reshape(R, Dm)
    af = actions.reshape(R, Dm)

    mesh = plsc.VectorSubcoreMesh(core_axis_name="c", subcore_axis_name="s")
    f32 = jnp.float32
    sc_call = functools.partial(
        pl.kernel,
        mesh=mesh,
        out_type=[jax.ShapeDtypeStruct((R, Dm), f32)] * 3,
        scratch_types=[
            pltpu.VMEM((3, Dm), f32),
            pltpu.VMEM((CR, Dm), f32),
            pltpu.VMEM((CR, Dm), f32),
            pltpu.VMEM((CR, Dm), f32),
            pltpu.VMEM((CR, Dm), f32),
            pltpu.SemaphoreType.DMA,
            pltpu.SemaphoreType.DMA,
            pltpu.SemaphoreType.DMA,
            pltpu.SemaphoreType.DMA,
        ],
    )(_sc_kernel_body)
    out = sc_call(lf, ff, af, emb_weight)
    return tuple(o.reshape(B, S, Dm) for o in out)


# speedup vs baseline: 3.1360x; 1.5719x over previous
import functools

import jax
import jax.numpy as jnp
from jax import lax
from jax.experimental import pallas as pl
from jax.experimental.pallas import tpu as pltpu
from jax.experimental.pallas import tpu_sc as plsc

D = 1024
L = 16
NSLICE = D // L
NC, NS = 2, 16
NW = NC * NS
R = 8192
ROWS_PER_W = R // NW
CR = 16
NCH = ROWS_PER_W // CR


def _sc_body(actions_hbm, emb_hbm, out_a,
             emb_v, in0, in1, ou0, ou1,
             si0, si1, so0, so1):
    wid = lax.axis_index("s") * NC + lax.axis_index("c")
    base = wid * ROWS_PER_W

    pltpu.sync_copy(emb_hbm, emb_v)

    in_bufs = (in0, in1)
    out_bufs = (ou0, ou1)
    in_sems = (si0, si1)
    out_sems = (so0, so1)

    def start_in(c, b):
        pltpu.make_async_copy(
            actions_hbm.at[pl.ds(base + c * CR, CR)], in_bufs[b], in_sems[b]
        ).start()

    def wait_in(b):
        pltpu.make_async_copy(
            actions_hbm.at[pl.ds(base, CR)], in_bufs[b], in_sems[b]
        ).wait()

    def start_out(c, b):
        pltpu.make_async_copy(
            out_bufs[b], out_a.at[pl.ds(base + c * CR, CR)], out_sems[b]
        ).start()

    def wait_out(b):
        pltpu.make_async_copy(
            out_bufs[b], out_a.at[pl.ds(base, CR)], out_sems[b]
        ).wait()

    def compute(b):
        ib, ob = in_bufs[b], out_bufs[b]
        GJ = 16
        for g in range(NSLICE // GJ):
            embs = [emb_v[2, pl.ds((g * GJ + k) * L, L)] for k in range(GJ)]

            @plsc.parallel_loop(0, CR, unroll=4)
            def _row(r):
                for k in range(GJ):
                    sl = pl.ds((g * GJ + k) * L, L)
                    ob[r, sl] = ib[r, sl] + embs[k]

    start_in(0, 0)
    start_in(1, 1)

    def pair_body(p, carry):
        for b in range(2):
            c = 2 * p + b
            wait_in(b)

            @pl.when(p > 0)
            def _():
                wait_out(b)

            compute(b)
            start_out(c, b)
            cn = jnp.minimum(c + 2, NCH - 1)
            start_in(cn, b)
        return carry

    lax.fori_loop(0, NCH // 2, pair_body, 0)
    wait_in(0)
    wait_in(1)
    wait_out(0)
    wait_out(1)


def _tc_body(lang_ref, frames_ref, emb_ref, out_l, out_f):
    out_l[...] = lang_ref[...] + emb_ref[0, :][None, :]
    out_f[...] = frames_ref[...] + emb_ref[1, :][None, :]


def kernel(lang, frames, actions, emb_weight):
    B, S, Dm = lang.shape
    lf = lang.reshape(R, Dm)
    ff = frames.reshape(R, Dm)
    af = actions.reshape(R, Dm)
    f32 = jnp.float32

    mesh = plsc.VectorSubcoreMesh(core_axis_name="c", subcore_axis_name="s")
    sc_call = functools.partial(
        pl.kernel,
        mesh=mesh,
        out_type=jax.ShapeDtypeStruct((R, Dm), f32),
        scratch_types=[
            pltpu.VMEM((3, Dm), f32),
            pltpu.VMEM((CR, Dm), f32),
            pltpu.VMEM((CR, Dm), f32),
            pltpu.VMEM((CR, Dm), f32),
            pltpu.VMEM((CR, Dm), f32),
            pltpu.SemaphoreType.DMA,
            pltpu.SemaphoreType.DMA,
            pltpu.SemaphoreType.DMA,
            pltpu.SemaphoreType.DMA,
        ],
    )(_sc_body)
    out_a = sc_call(af, emb_weight)

    BR = 512
    spec = pl.BlockSpec((BR, Dm), lambda i: (i, 0))
    emb_spec = pl.BlockSpec((3, Dm), lambda i: (0, 0))
    out_l, out_f = pl.pallas_call(
        _tc_body,
        grid=(R // BR,),
        in_specs=[spec, spec, emb_spec],
        out_specs=[spec, spec],
        out_shape=[jax.ShapeDtypeStruct((R, Dm), f32)] * 2,
    )(lf, ff, emb_weight)

    return (out_l.reshape(B, S, Dm), out_f.reshape(B, S, Dm),
            out_a.reshape(B, S, Dm))


# speedup vs baseline: 3.1830x; 1.0150x over previous
import functools

import jax
import jax.numpy as jnp
from jax import lax
from jax.experimental import pallas as pl
from jax.experimental.pallas import tpu as pltpu
from jax.experimental.pallas import tpu_sc as plsc

D = 1024
L = 16
NSLICE = D // L
NC, NS = 2, 16
NW = NC * NS
R = 8192
ROWS_PER_W = R // NW
CR = 32
NCH = ROWS_PER_W // CR


def _sc_body(actions_hbm, emb_hbm, out_a,
             emb_v, buf0, buf1,
             si0, si1, so0, so1):
    wid = lax.axis_index("s") * NC + lax.axis_index("c")
    base = wid * ROWS_PER_W

    pltpu.sync_copy(emb_hbm, emb_v)

    bufs = (buf0, buf1)
    in_sems = (si0, si1)
    out_sems = (so0, so1)

    def start_in(c, b):
        pltpu.make_async_copy(
            actions_hbm.at[pl.ds(base + c * CR, CR)], bufs[b], in_sems[b]
        ).start()

    def wait_in(b):
        pltpu.make_async_copy(
            actions_hbm.at[pl.ds(base, CR)], bufs[b], in_sems[b]
        ).wait()

    def start_out(c, b):
        pltpu.make_async_copy(
            bufs[b], out_a.at[pl.ds(base + c * CR, CR)], out_sems[b]
        ).start()

    def wait_out(b):
        pltpu.make_async_copy(
            bufs[b], out_a.at[pl.ds(base, CR)], out_sems[b]
        ).wait()

    def compute(b):
        buf = bufs[b]
        GJ = 16
        for g in range(NSLICE // GJ):
            embs = [emb_v[2, pl.ds((g * GJ + k) * L, L)] for k in range(GJ)]

            @plsc.parallel_loop(0, CR, unroll=4)
            def _row(r):
                for k in range(GJ):
                    sl = pl.ds((g * GJ + k) * L, L)
                    buf[r, sl] = buf[r, sl] + embs[k]

    start_in(0, 0)
    start_in(1, 1)

    def pair_body(p, carry):
        for b in range(2):
            c = 2 * p + b
            wait_in(b)
            compute(b)
            start_out(c, b)

            @pl.when(p + 1 < NCH // 2)
            def _():
                wait_out(b)
                start_in(c + 2, b)

        return carry

    lax.fori_loop(0, NCH // 2, pair_body, 0)
    wait_out(0)
    wait_out(1)


def _tc_body(lang_ref, frames_ref, emb_ref, out_l, out_f):
    out_l[...] = lang_ref[...] + emb_ref[0, :][None, :]
    out_f[...] = frames_ref[...] + emb_ref[1, :][None, :]


def kernel(lang, frames, actions, emb_weight):
    B, S, Dm = lang.shape
    lf = lang.reshape(R, Dm)
    ff = frames.reshape(R, Dm)
    af = actions.reshape(R, Dm)
    f32 = jnp.float32

    mesh = plsc.VectorSubcoreMesh(core_axis_name="c", subcore_axis_name="s")
    sc_call = functools.partial(
        pl.kernel,
        mesh=mesh,
        out_type=jax.ShapeDtypeStruct((R, Dm), f32),
        scratch_types=[
            pltpu.VMEM((3, Dm), f32),
            pltpu.VMEM((CR, Dm), f32),
            pltpu.VMEM((CR, Dm), f32),
            pltpu.SemaphoreType.DMA,
            pltpu.SemaphoreType.DMA,
            pltpu.SemaphoreType.DMA,
            pltpu.SemaphoreType.DMA,
        ],
    )(_sc_body)
    out_a = sc_call(af, emb_weight)

    BR = 512
    spec = pl.BlockSpec((BR, Dm), lambda i: (i, 0))
    emb_spec = pl.BlockSpec((3, Dm), lambda i: (0, 0))
    out_l, out_f = pl.pallas_call(
        _tc_body,
        grid=(R // BR,),
        in_specs=[spec, spec, emb_spec],
        out_specs=[spec, spec],
        out_shape=[jax.ShapeDtypeStruct((R, Dm), f32)] * 2,
    )(lf, ff, emb_weight)

    return (out_l.reshape(B, S, Dm), out_f.reshape(B, S, Dm),
            out_a.reshape(B, S, Dm))


# speedup vs baseline: 3.2729x; 1.0283x over previous
import functools

import jax
import jax.numpy as jnp
from jax import lax
from jax.experimental import pallas as pl
from jax.experimental.pallas import tpu as pltpu
from jax.experimental.pallas import tpu_sc as plsc

D = 1024
L = 16
NSLICE = D // L
NC, NS = 2, 16
NW = NC * NS
R = 8192
ROWS_PER_W = R // NW
CR = 32
NCH = ROWS_PER_W // CR


def _sc_body(actions_hbm, emb_hbm, out_a,
             emb_v, buf0, buf1,
             si0, si1, so0, so1):
    wid = lax.axis_index("s") * NC + lax.axis_index("c")
    base = wid * ROWS_PER_W

    pltpu.sync_copy(emb_hbm, emb_v)

    bufs = (buf0, buf1)
    in_sems = (si0, si1)
    out_sems = (so0, so1)

    def start_in(c, b):
        pltpu.make_async_copy(
            actions_hbm.at[pl.ds(base + c * CR, CR)], bufs[b], in_sems[b]
        ).start()

    def wait_in(b):
        pltpu.make_async_copy(
            actions_hbm.at[pl.ds(base, CR)], bufs[b], in_sems[b]
        ).wait()

    def start_out(c, b):
        pltpu.make_async_copy(
            bufs[b], out_a.at[pl.ds(base + c * CR, CR)], out_sems[b]
        ).start()

    def wait_out(b):
        pltpu.make_async_copy(
            bufs[b], out_a.at[pl.ds(base, CR)], out_sems[b]
        ).wait()

    def compute(b):
        buf = bufs[b]
        GJ = 16
        for g in range(NSLICE // GJ):
            embs = [emb_v[2, pl.ds((g * GJ + k) * L, L)] for k in range(GJ)]

            @plsc.parallel_loop(0, CR, unroll=4)
            def _row(r):
                for k in range(GJ):
                    sl = pl.ds((g * GJ + k) * L, L)
                    buf[r, sl] = buf[r, sl] + embs[k]

    start_in(0, 0)
    start_in(1, 1)

    def pair_body(p, carry):
        for b in range(2):
            c = 2 * p + b
            wait_in(b)
            compute(b)
            start_out(c, b)

            @pl.when(p + 1 < NCH // 2)
            def _():
                wait_out(b)
                start_in(c + 2, b)

        return carry

    lax.fori_loop(0, NCH // 2, pair_body, 0)
    wait_out(0)
    wait_out(1)


def _tc_body(lang_ref, frames_ref, emb_ref, out_l, out_f):
    out_l[...] = lang_ref[...] + emb_ref[0, :][None, :]
    out_f[...] = frames_ref[...] + emb_ref[1, :][None, :]


def kernel(lang, frames, actions, emb_weight):
    B, S, Dm = lang.shape
    lf = lang.reshape(R, Dm)
    ff = frames.reshape(R, Dm)
    af = actions.reshape(R, Dm)
    f32 = jnp.float32

    mesh = plsc.VectorSubcoreMesh(core_axis_name="c", subcore_axis_name="s")
    sc_call = functools.partial(
        pl.kernel,
        mesh=mesh,
        out_type=jax.ShapeDtypeStruct((R, Dm), f32),
        scratch_types=[
            pltpu.VMEM((3, Dm), f32),
            pltpu.VMEM((CR, Dm), f32),
            pltpu.VMEM((CR, Dm), f32),
            pltpu.SemaphoreType.DMA,
            pltpu.SemaphoreType.DMA,
            pltpu.SemaphoreType.DMA,
            pltpu.SemaphoreType.DMA,
        ],
    )(_sc_body)
    out_a = sc_call(af, emb_weight)

    BR = 1024
    spec = pl.BlockSpec((BR, Dm), lambda i: (i, 0))
    emb_spec = pl.BlockSpec((3, Dm), lambda i: (0, 0))
    out_l, out_f = pl.pallas_call(
        _tc_body,
        grid=(R // BR,),
        in_specs=[spec, spec, emb_spec],
        out_specs=[spec, spec],
        out_shape=[jax.ShapeDtypeStruct((R, Dm), f32)] * 2,
    )(lf, ff, emb_weight)

    return (out_l.reshape(B, S, Dm), out_f.reshape(B, S, Dm),
            out_a.reshape(B, S, Dm))
